# baseline (speedup 1.0000x reference)
import functools

import jax
import jax.numpy as jnp
from jax import lax
from jax.experimental import pallas as pl
from jax.experimental.pallas import tpu as pltpu
from jax.experimental.pallas import tpu_sc as plsc

DIM = 256
N_ATOMS = 1024
ROWS = 36864
ROW_TILE = 256
GRID = ROWS // ROW_TILE


def _argmax_body(x_ref, atoms_ref, idx_ref, an_scratch):
    @pl.when(pl.program_id(0) == 0)
    def _():
        a = atoms_ref[...]
        n = jnp.sqrt(jnp.sum(a * a, axis=1, keepdims=True))
        an_scratch[...] = a / jnp.maximum(n, 1e-12)

    x = x_ref[...]
    logits = lax.dot_general(
        x, an_scratch[...],
        dimension_numbers=(((1,), (1,)), ((), ())),
        preferred_element_type=jnp.float32,
    )
    m = jnp.max(logits, axis=1, keepdims=True)
    ids = lax.broadcasted_iota(jnp.int32, logits.shape, 1)
    cand = jnp.where(logits == m, ids, jnp.int32(2**30))
    idx_ref[...] = jnp.min(cand, axis=1).reshape(1, 1, ROW_TILE)


def _compute_argmax(x, atoms):
    idx3 = pl.pallas_call(
        _argmax_body,
        grid=(GRID,),
        in_specs=[
            pl.BlockSpec((ROW_TILE, DIM), lambda t: (t, 0)),
            pl.BlockSpec((N_ATOMS, DIM), lambda t: (0, 0)),
        ],
        out_specs=pl.BlockSpec((1, 1, ROW_TILE), lambda t: (t, 0, 0)),
        out_shape=jax.ShapeDtypeStruct((GRID, 1, ROW_TILE), jnp.int32),
        scratch_shapes=[pltpu.VMEM((N_ATOMS, DIM), jnp.float32)],
    )(x, atoms)
    return idx3.reshape(ROWS)


_NW = 32
_B_PER_W = ROWS // _NW
_CHUNK = 128
_NCHUNK = _B_PER_W // _CHUNK


def _sc_gather_body(table_hbm, idx_hbm, out_hbm, idx_v, rows_v, sem):
    wid = lax.axis_index("s") * 2 + lax.axis_index("c")
    base = wid * _B_PER_W
    pltpu.sync_copy(idx_hbm.at[pl.ds(base, _B_PER_W)], idx_v)
    for ch in range(_NCHUNK):
        cpy = pltpu.async_copy(
            table_hbm.at[idx_v.at[pl.ds(ch * _CHUNK, _CHUNK)]],
            rows_v, sem)
        cpy.wait()
        pltpu.sync_copy(rows_v, out_hbm.at[pl.ds(base + ch * _CHUNK, _CHUNK)])


@functools.cache
def _sc_gather():
    return pl.kernel(
        _sc_gather_body,
        out_type=jax.ShapeDtypeStruct((ROWS, DIM), jnp.float32),
        mesh=plsc.VectorSubcoreMesh(core_axis_name="c", subcore_axis_name="s"),
        scratch_types=[
            pltpu.VMEM((_B_PER_W,), jnp.int32),
            pltpu.VMEM((_CHUNK, DIM), jnp.float32),
            pltpu.SemaphoreType.DMA,
        ],
    )


def kernel(embeddings, atoms, c_ids):
    del c_ids
    x = embeddings.reshape(ROWS, DIM)
    idx_nat = _compute_argmax(x, atoms)
    v = idx_nat.reshape(1024, 36)
    idx_perm = jnp.concatenate([
        v[:, :4].T.reshape(-1),
        v[:, 4:20].reshape(-1),
        v[:, 20:36].reshape(-1),
    ])
    return _sc_gather()(atoms, idx_perm)


# speedup vs baseline: 1.6107x; 1.6107x over previous
import functools

import jax
import jax.numpy as jnp
from jax import lax
from jax.experimental import pallas as pl
from jax.experimental.pallas import tpu as pltpu
from jax.experimental.pallas import tpu_sc as plsc

DIM = 256
N_ATOMS = 1024
ROWS = 36864
L_TOT = 36
B = 1024
BT = 32
GRID = B // BT


def _argmax_body(x_ref, atoms_ref, idx_ref, an_scratch):
    @pl.when(pl.program_id(0) == 0)
    def _():
        a = atoms_ref[...]
        n = jnp.sqrt(jnp.sum(a * a, axis=1, keepdims=True))
        an_scratch[...] = a / jnp.maximum(n, 1e-12)

    x = x_ref[...].reshape(BT * L_TOT, DIM)
    logits = lax.dot_general(
        x, an_scratch[...],
        dimension_numbers=(((1,), (1,)), ((), ())),
        preferred_element_type=jnp.float32,
    )
    idx = jnp.argmax(logits, axis=1).astype(jnp.int32)
    idx_ref[...] = idx.reshape(BT, L_TOT)


def _compute_argmax(embeddings, atoms):
    return pl.pallas_call(
        _argmax_body,
        grid=(GRID,),
        in_specs=[
            pl.BlockSpec((BT, L_TOT, DIM), lambda t: (t, 0, 0)),
            pl.BlockSpec((N_ATOMS, DIM), lambda t: (0, 0)),
        ],
        out_specs=pl.BlockSpec((BT, L_TOT), lambda t: (t, 0)),
        out_shape=jax.ShapeDtypeStruct((B, L_TOT), jnp.int32),
        scratch_shapes=[pltpu.VMEM((N_ATOMS, DIM), jnp.float32)],
    )(embeddings, atoms)


_NW = 32
_B_PER_W = ROWS // _NW
_CHUNK = 128
_NCHUNK = _B_PER_W // _CHUNK


def _sc_gather_body(table_hbm, idx_hbm, out_hbm, idx_v, rows0, rows1, sem0,
                    sem1):
    wid = lax.axis_index("s") * 2 + lax.axis_index("c")
    base = wid * _B_PER_W
    pltpu.sync_copy(idx_hbm.at[pl.ds(base, _B_PER_W)], idx_v)
    bufs, sems = (rows0, rows1), (sem0, sem1)
    cps = [pltpu.async_copy(table_hbm.at[idx_v.at[pl.ds(0, _CHUNK)]],
                            bufs[0], sems[0])]
    for ch in range(_NCHUNK):
        if ch + 1 < _NCHUNK:
            cps.append(pltpu.async_copy(
                table_hbm.at[idx_v.at[pl.ds((ch + 1) * _CHUNK, _CHUNK)]],
                bufs[(ch + 1) % 2], sems[(ch + 1) % 2]))
        cps[ch].wait()
        pltpu.sync_copy(bufs[ch % 2],
                        out_hbm.at[pl.ds(base + ch * _CHUNK, _CHUNK)])


@functools.cache
def _sc_gather():
    return pl.kernel(
        _sc_gather_body,
        out_type=jax.ShapeDtypeStruct((ROWS, DIM), jnp.float32),
        mesh=plsc.VectorSubcoreMesh(core_axis_name="c", subcore_axis_name="s"),
        scratch_types=[
            pltpu.VMEM((_B_PER_W,), jnp.int32),
            pltpu.VMEM((_CHUNK, DIM), jnp.float32),
            pltpu.VMEM((_CHUNK, DIM), jnp.float32),
            pltpu.SemaphoreType.DMA,
            pltpu.SemaphoreType.DMA,
        ],
    )


def kernel(embeddings, atoms, c_ids):
    del c_ids
    v = _compute_argmax(embeddings, atoms)
    idx_perm = jnp.concatenate([
        v[:, :4].T.reshape(-1),
        v[:, 4:20].reshape(-1),
        v[:, 20:36].reshape(-1),
    ])
    return _sc_gather()(atoms, idx_perm)


# speedup vs baseline: 2.4888x; 1.5451x over previous
import functools

import jax
import jax.numpy as jnp
from jax import lax
from jax.experimental import pallas as pl
from jax.experimental.pallas import tpu as pltpu
from jax.experimental.pallas import tpu_sc as plsc

DIM = 256
N_ATOMS = 1024
ROWS = 36864
L_TOT = 36
B = 1024
BT = 32
GRID = B // BT


def _argmax_body(x_ref, atoms_ref, idx_ref, an_scratch):
    @pl.when(pl.program_id(0) == 0)
    def _():
        a = atoms_ref[...]
        n = jnp.sqrt(jnp.sum(a * a, axis=1, keepdims=True))
        an_scratch[...] = a / jnp.maximum(n, 1e-12)

    x = x_ref[...].reshape(L_TOT * BT, DIM)
    logits = lax.dot_general(
        x, an_scratch[...],
        dimension_numbers=(((1,), (1,)), ((), ())),
        preferred_element_type=jnp.float32,
    )
    idx = jnp.argmax(logits, axis=1).astype(jnp.int32)
    idx_ref[...] = idx.reshape(1, L_TOT, BT)


def _compute_argmax(embeddings, atoms):
    xt = embeddings.transpose(1, 0, 2)
    return pl.pallas_call(
        _argmax_body,
        grid=(GRID,),
        in_specs=[
            pl.BlockSpec((L_TOT, BT, DIM), lambda t: (0, t, 0)),
            pl.BlockSpec((N_ATOMS, DIM), lambda t: (0, 0)),
        ],
        out_specs=pl.BlockSpec((1, L_TOT, BT), lambda t: (t, 0, 0)),
        out_shape=jax.ShapeDtypeStruct((GRID, L_TOT, BT), jnp.int32),
        scratch_shapes=[pltpu.VMEM((N_ATOMS, DIM), jnp.float32)],
    )(xt, atoms)


_NW = 32
_B_PER_W = ROWS // _NW
_CHUNK = 128
_NCHUNK = _B_PER_W // _CHUNK


def _sc_gather_body(table_hbm, idx_hbm, out_hbm, idx_v, b0, b1, b2,
                    g0, g1, g2, s0, s1, s2):
    bufs, gsem, ssem = (b0, b1, b2), (g0, g1, g2), (s0, s1, s2)
    wid = lax.axis_index("s") * 2 + lax.axis_index("c")
    base = wid * _B_PER_W
    pltpu.sync_copy(idx_hbm.at[pl.ds(base, _B_PER_W)], idx_v)

    def gather(ch):
        return pltpu.async_copy(
            table_hbm.at[idx_v.at[pl.ds(ch * _CHUNK, _CHUNK)]],
            bufs[ch % 3], gsem[ch % 3])

    gets = {0: gather(0), 1: gather(1)}
    scs = {}
    for ch in range(_NCHUNK):
        nb = ch + 2
        if nb < _NCHUNK:
            if nb >= 3:
                scs[nb - 3].wait()
            gets[nb] = gather(nb)
        gets[ch].wait()
        scs[ch] = pltpu.async_copy(
            bufs[ch % 3], out_hbm.at[pl.ds(base + ch * _CHUNK, _CHUNK)],
            ssem[ch % 3])
    for ch in range(max(0, _NCHUNK - 3), _NCHUNK):
        scs[ch].wait()


@functools.cache
def _sc_gather():
    return pl.kernel(
        _sc_gather_body,
        out_type=jax.ShapeDtypeStruct((ROWS, DIM), jnp.float32),
        mesh=plsc.VectorSubcoreMesh(core_axis_name="c", subcore_axis_name="s"),
        scratch_types=[
            pltpu.VMEM((_B_PER_W,), jnp.int32),
            pltpu.VMEM((_CHUNK, DIM), jnp.float32),
            pltpu.VMEM((_CHUNK, DIM), jnp.float32),
            pltpu.VMEM((_CHUNK, DIM), jnp.float32),
            pltpu.SemaphoreType.DMA,
            pltpu.SemaphoreType.DMA,
            pltpu.SemaphoreType.DMA,
            pltpu.SemaphoreType.DMA,
            pltpu.SemaphoreType.DMA,
            pltpu.SemaphoreType.DMA,
        ],
    )


def kernel(embeddings, atoms, c_ids):
    del c_ids
    idx3 = _compute_argmax(embeddings, atoms)
    v = idx3.transpose(1, 0, 2).reshape(L_TOT, B)
    idx_perm = jnp.concatenate([
        v[:4].reshape(-1),
        v[4:20].T.reshape(-1),
        v[20:36].T.reshape(-1),
    ])
    return _sc_gather()(atoms, idx_perm)


# speedup vs baseline: 2.4900x; 1.0005x over previous
import functools

import jax
import jax.numpy as jnp
from jax import lax
from jax.experimental import pallas as pl
from jax.experimental.pallas import tpu as pltpu
from jax.experimental.pallas import tpu_sc as plsc

DIM = 256
N_ATOMS = 1024
ROWS = 36864
L_TOT = 36
B = 1024
BT = 32
GRID = B // BT


def _argmax_body(x_ref, atoms_ref, idx_ref, an_scratch):
    @pl.when(pl.program_id(0) == 0)
    def _():
        a = atoms_ref[...]
        n = jnp.sqrt(jnp.sum(a * a, axis=1, keepdims=True))
        an_scratch[...] = a / jnp.maximum(n, 1e-12)

    x = x_ref[...].reshape(L_TOT * BT, DIM)
    logits = lax.dot_general(
        x, an_scratch[...],
        dimension_numbers=(((1,), (1,)), ((), ())),
        preferred_element_type=jnp.float32,
    )
    idx = jnp.argmax(logits, axis=1).astype(jnp.int32)
    idx_ref[...] = idx.reshape(1, L_TOT, BT)


def _compute_argmax(embeddings, atoms):
    xt = embeddings.transpose(1, 0, 2)
    return pl.pallas_call(
        _argmax_body,
        grid=(GRID,),
        in_specs=[
            pl.BlockSpec((L_TOT, BT, DIM), lambda t: (0, t, 0)),
            pl.BlockSpec((N_ATOMS, DIM), lambda t: (0, 0)),
        ],
        out_specs=pl.BlockSpec((1, L_TOT, BT), lambda t: (t, 0, 0)),
        out_shape=jax.ShapeDtypeStruct((GRID, L_TOT, BT), jnp.int32),
        scratch_shapes=[pltpu.VMEM((N_ATOMS, DIM), jnp.float32)],
    )(xt, atoms)


_NW = 32
_B_PER_W = ROWS // _NW
_CHUNK = 128
_NCHUNK = _B_PER_W // _CHUNK


def _sc_gather_body(table_hbm, idx_hbm, out_hbm, idx_v, b0, b1, b2,
                    g0, g1, g2, s0, s1, s2):
    bufs, gsem, ssem = (b0, b1, b2), (g0, g1, g2), (s0, s1, s2)
    wid = lax.axis_index("s") * 2 + lax.axis_index("c")
    base = wid * _B_PER_W
    pltpu.sync_copy(idx_hbm.at[wid], idx_v)

    def gather(ch):
        return pltpu.async_copy(
            table_hbm.at[idx_v.at[ch]],
            bufs[ch % 3], gsem[ch % 3])

    gets = {0: gather(0), 1: gather(1)}
    scs = {}
    for ch in range(_NCHUNK):
        nb = ch + 2
        if nb < _NCHUNK:
            if nb >= 3:
                scs[nb - 3].wait()
            gets[nb] = gather(nb)
        gets[ch].wait()
        scs[ch] = pltpu.async_copy(
            bufs[ch % 3], out_hbm.at[pl.ds(base + ch * _CHUNK, _CHUNK)],
            ssem[ch % 3])
    for ch in range(max(0, _NCHUNK - 3), _NCHUNK):
        scs[ch].wait()


@functools.cache
def _sc_gather():
    return pl.kernel(
        _sc_gather_body,
        out_type=jax.ShapeDtypeStruct((ROWS, DIM), jnp.float32),
        mesh=plsc.VectorSubcoreMesh(core_axis_name="c", subcore_axis_name="s"),
        scratch_types=[
            pltpu.VMEM((_NCHUNK, _CHUNK), jnp.int32),
            pltpu.VMEM((_CHUNK, DIM), jnp.float32),
            pltpu.VMEM((_CHUNK, DIM), jnp.float32),
            pltpu.VMEM((_CHUNK, DIM), jnp.float32),
            pltpu.SemaphoreType.DMA,
            pltpu.SemaphoreType.DMA,
            pltpu.SemaphoreType.DMA,
            pltpu.SemaphoreType.DMA,
            pltpu.SemaphoreType.DMA,
            pltpu.SemaphoreType.DMA,
        ],
    )


def kernel(embeddings, atoms, c_ids):
    del c_ids
    idx3 = _compute_argmax(embeddings, atoms)
    v = idx3.transpose(1, 0, 2).reshape(L_TOT, B)
    idx_perm = jnp.concatenate([
        v[:4].reshape(-1),
        v[4:20].T.reshape(-1),
        v[20:36].T.reshape(-1),
    ])
    return _sc_gather()(atoms, idx_perm.reshape(_NW, _NCHUNK, _CHUNK))
